# baseline (speedup 1.0000x reference)
import jax
import jax.numpy as jnp
from jax.experimental import pallas as pl
from jax.experimental.pallas import tpu as pltpu

_B, _C, _H, _W = 32, 3, 224, 224
_P = 16
_N = (_H // _P) * (_W // _P)
_D = 768
_DFF = 3072
_E = 3
_NCLS = 1000
_SPB = 4
_BLK = _SPB * _N
_G = _B // _SPB
_EPS = 1e-6


def _fused_kernel(p_ref, wpe_ref, bpe_ref, gblk_ref, bblk_ref,
                  w1_ref, b1_ref, w2_ref, b2_ref, seg_ref, a_ref, bs_ref,
                  gbase_ref, bbase_ref, nw_ref, nb_ref, wh_ref, bh_ref,
                  out_ref, pooled_acc, scout_acc):
    i = pl.program_id(0)
    pb = p_ref[:]

    scout_cols = [jnp.sum(pb * a_ref[e], axis=1, keepdims=True)
                  for e in range(_E)]
    scout_tok = jnp.concatenate(scout_cols, axis=1)

    bf16 = jnp.bfloat16
    h = jnp.dot(pb.astype(bf16), wpe_ref[:], preferred_element_type=jnp.float32)
    h = h + bpe_ref[0, :]
    mu = jnp.mean(h, axis=-1, keepdims=True)
    var = jnp.mean((h - mu) * (h - mu), axis=-1, keepdims=True)
    hn = (h - mu) * jax.lax.rsqrt(var + _EPS) * gblk_ref[0, :] + bblk_ref[0, :]
    t = jnp.dot(hn.astype(bf16), w1_ref[:], preferred_element_type=jnp.float32)
    t = jax.nn.gelu(t + b1_ref[0, :])
    h = h + jnp.dot(t.astype(bf16), w2_ref[:], preferred_element_type=jnp.float32)
    h = h + b2_ref[0, :]

    pooled8 = jnp.dot(seg_ref[:], h, preferred_element_type=jnp.float32)
    scout8 = jnp.dot(seg_ref[:], scout_tok, preferred_element_type=jnp.float32)
    pooled_acc[pl.ds(8 * i, 8), :] = pooled8
    scout_acc[pl.ds(8 * i, 8), :] = scout8 * jnp.float32(_N)

    @pl.when(i == _G - 1)
    def _():
        gate = jax.nn.softmax(scout_acc[:] + bs_ref[0, :], axis=-1)
        pa = pooled_acc[:]
        mu2 = jnp.mean(pa, axis=-1, keepdims=True)
        var2 = jnp.mean((pa - mu2) * (pa - mu2), axis=-1, keepdims=True)
        xn = (pa - mu2) * jax.lax.rsqrt(var2 + _EPS)
        geff = gbase_ref[0, :] + jnp.dot(gate, nw_ref[:],
                                         preferred_element_type=jnp.float32)
        beff = bbase_ref[0, :] + jnp.dot(gate, nb_ref[:],
                                         preferred_element_type=jnp.float32)
        feat = xn * geff + beff
        logits = jnp.dot(feat, wh_ref[:], preferred_element_type=jnp.float32)
        out_ref[:] = logits + bh_ref[0, :]


def kernel(x, W_s, b_s, W_pe, b_pe, g_blk, b_blk, W1, b1, W2, b2,
           g_base, b_base, norms_w, norms_b, W_h, b_h):
    f32 = jnp.float32

    p = x.reshape(_B, _C, _H // _P, _P, _W // _P, _P)
    p = p.transpose(0, 2, 4, 1, 3, 5).reshape(_B * _N, _P * _P * _C)

    n = jnp.arange(_N)[:, None]
    f = jnp.arange(_P * _P * _C)[None, :]
    c = f // (_P * _P)
    pi = (f % (_P * _P)) // _P
    pj = f % _P
    hh = (n // (_W // _P)) * _P + pi
    ww = (n % (_W // _P)) * _P + pj
    idx = c * 64 + (hh // 28) * 8 + (ww // 28)
    amap = (W_s.T[:, idx] * (1.0 / 784.0)).astype(f32)
    amap = jnp.tile(amap, (1, _SPB, 1))

    row = jax.lax.broadcasted_iota(jnp.int32, (8, _BLK), 0)
    col = jax.lax.broadcasted_iota(jnp.int32, (8, _BLK), 1)
    seg = jnp.where(col // _N == row, 1.0 / _N, 0.0).astype(f32)

    full2 = lambda shp: pl.BlockSpec(shp, lambda i: (0, 0))
    logits64 = pl.pallas_call(
        _fused_kernel,
        grid=(_G,),
        in_specs=[
            pl.BlockSpec((_BLK, _D), lambda i: (i, 0)),
            full2((_D, _D)),
            full2((1, _D)),
            full2((1, _D)),
            full2((1, _D)),
            full2((_D, _DFF)),
            full2((1, _DFF)),
            full2((_DFF, _D)),
            full2((1, _D)),
            full2((8, _BLK)),
            pl.BlockSpec((_E, _BLK, _D), lambda i: (0, 0, 0)),
            full2((1, _E)),
            full2((1, _D)),
            full2((1, _D)),
            full2((_E, _D)),
            full2((_E, _D)),
            full2((_D, _NCLS)),
            full2((1, _NCLS)),
        ],
        out_specs=full2((_G * 8, _NCLS)),
        out_shape=jax.ShapeDtypeStruct((_G * 8, _NCLS), f32),
        scratch_shapes=[pltpu.VMEM((_G * 8, _D), f32),
                        pltpu.VMEM((_G * 8, _E), f32)],
    )(p, W_pe.astype(jnp.bfloat16), b_pe.reshape(1, _D),
      g_blk.reshape(1, _D), b_blk.reshape(1, _D),
      W1.astype(jnp.bfloat16), b1.reshape(1, _DFF), W2.astype(jnp.bfloat16),
      b2.reshape(1, _D), seg, amap,
      b_s.reshape(1, _E), g_base.reshape(1, _D), b_base.reshape(1, _D),
      norms_w, norms_b, W_h, b_h.reshape(1, _NCLS))

    logits = logits64.reshape(_G, 8, _NCLS)[:, :_SPB].reshape(_B, _NCLS)
    return logits


# speedup vs baseline: 2.3066x; 2.3066x over previous
import jax
import jax.numpy as jnp
from jax.experimental import pallas as pl
from jax.experimental.pallas import tpu as pltpu

_B, _C, _H, _W = 32, 3, 224, 224
_P = 16
_N = (_H // _P) * (_W // _P)
_D = 768
_DFF = 3072
_E = 3
_NCLS = 1000
_SPB = 4
_BLK = _SPB * _N
_G = _B // _SPB
_EPS = 1e-6


def _fused_kernel(p_ref, xf_ref, wpe_ref, bpe_ref, gblk_ref, bblk_ref,
                  w1_ref, b1_ref, w2_ref, b2_ref, seg_ref, wsx_ref, bs_ref,
                  gbase_ref, bbase_ref, nw_ref, nb_ref, wh_ref, bh_ref,
                  out_ref, pooled_acc, scout_acc):
    i = pl.program_id(0)
    pb = p_ref[:]

    xf = xf_ref[0]
    scout_cols = [jnp.sum(xf * wsx_ref[e, :][None, :], axis=1, keepdims=True)
                  for e in range(_E)]
    scout4 = jnp.concatenate(scout_cols, axis=1)

    bf16 = jnp.bfloat16
    h = jnp.dot(pb.astype(bf16), wpe_ref[:], preferred_element_type=jnp.float32)
    h = h + bpe_ref[0, :]
    mu = jnp.mean(h, axis=-1, keepdims=True)
    var = jnp.mean((h - mu) * (h - mu), axis=-1, keepdims=True)
    hn = (h - mu) * jax.lax.rsqrt(var + _EPS) * gblk_ref[0, :] + bblk_ref[0, :]
    t = jnp.dot(hn.astype(bf16), w1_ref[:], preferred_element_type=jnp.float32)
    t = jax.nn.gelu(t + b1_ref[0, :])
    h = h + jnp.dot(t.astype(bf16), w2_ref[:], preferred_element_type=jnp.float32)
    h = h + b2_ref[0, :]

    pooled8 = jnp.dot(seg_ref[:], h, preferred_element_type=jnp.float32)
    pooled_acc[pl.ds(8 * i, 8), :] = pooled8
    scout_acc[pl.ds(8 * i, 8), :] = jnp.concatenate(
        [scout4, jnp.zeros_like(scout4)], axis=0)

    @pl.when(i == _G - 1)
    def _():
        gate = jax.nn.softmax(scout_acc[:] + bs_ref[0, :], axis=-1)
        pa = pooled_acc[:]
        mu2 = jnp.mean(pa, axis=-1, keepdims=True)
        var2 = jnp.mean((pa - mu2) * (pa - mu2), axis=-1, keepdims=True)
        xn = (pa - mu2) * jax.lax.rsqrt(var2 + _EPS)
        geff = gbase_ref[0, :] + jnp.dot(gate, nw_ref[:],
                                         preferred_element_type=jnp.float32)
        beff = bbase_ref[0, :] + jnp.dot(gate, nb_ref[:],
                                         preferred_element_type=jnp.float32)
        feat = xn * geff + beff
        logits = jnp.dot(feat, wh_ref[:], preferred_element_type=jnp.float32)
        out_ref[:] = logits + bh_ref[0, :]


def kernel(x, W_s, b_s, W_pe, b_pe, g_blk, b_blk, W1, b1, W2, b2,
           g_base, b_base, norms_w, norms_b, W_h, b_h):
    f32 = jnp.float32

    p = x.reshape(_B, _C, _H // _P, _P, _W // _P, _P)
    p = p.transpose(0, 2, 4, 1, 3, 5).reshape(_B * _N, _P * _P * _C)

    wsx = W_s.T.reshape(_E, _C, 8, 1, 8, 1) * (1.0 / 784.0)
    wsx = jnp.broadcast_to(wsx, (_E, _C, 8, 28, 8, 28))
    wsx = wsx.reshape(_E, _C * _H * _W).astype(f32)
    xf3 = x.reshape(_G, _SPB, _C * _H * _W)

    row = jax.lax.broadcasted_iota(jnp.int32, (8, _BLK), 0)
    col = jax.lax.broadcasted_iota(jnp.int32, (8, _BLK), 1)
    seg = jnp.where(col // _N == row, 1.0 / _N, 0.0).astype(f32)

    full2 = lambda shp: pl.BlockSpec(shp, lambda i: (0, 0))
    logits64 = pl.pallas_call(
        _fused_kernel,
        grid=(_G,),
        in_specs=[
            pl.BlockSpec((_BLK, _D), lambda i: (i, 0)),
            pl.BlockSpec((1, _SPB, _C * _H * _W),
                         lambda i: (i, 0, 0)),
            full2((_D, _D)),
            full2((1, _D)),
            full2((1, _D)),
            full2((1, _D)),
            full2((_D, _DFF)),
            full2((1, _DFF)),
            full2((_DFF, _D)),
            full2((1, _D)),
            full2((8, _BLK)),
            full2((_E, _C * _H * _W)),
            full2((1, _E)),
            full2((1, _D)),
            full2((1, _D)),
            full2((_E, _D)),
            full2((_E, _D)),
            full2((_D, _NCLS)),
            full2((1, _NCLS)),
        ],
        out_specs=full2((_G * 8, _NCLS)),
        out_shape=jax.ShapeDtypeStruct((_G * 8, _NCLS), f32),
        scratch_shapes=[pltpu.VMEM((_G * 8, _D), f32),
                        pltpu.VMEM((_G * 8, _E), f32)],
    )(p, xf3, W_pe.astype(jnp.bfloat16), b_pe.reshape(1, _D),
      g_blk.reshape(1, _D), b_blk.reshape(1, _D),
      W1.astype(jnp.bfloat16), b1.reshape(1, _DFF), W2.astype(jnp.bfloat16),
      b2.reshape(1, _D), seg, wsx,
      b_s.reshape(1, _E), g_base.reshape(1, _D), b_base.reshape(1, _D),
      norms_w, norms_b, W_h, b_h.reshape(1, _NCLS))

    logits = logits64.reshape(_G, 8, _NCLS)[:, :_SPB].reshape(_B, _NCLS)
    return logits


# speedup vs baseline: 2.4767x; 1.0737x over previous
import jax
import jax.numpy as jnp
from jax.experimental import pallas as pl
from jax.experimental.pallas import tpu as pltpu

_B, _C, _H, _W = 32, 3, 224, 224
_P = 16
_N = (_H // _P) * (_W // _P)
_D = 768
_DFF = 3072
_E = 3
_NCLS = 1000
_SPB = 4
_BLK = _SPB * _N
_G = _B // _SPB
_EPS = 1e-6


def _fused_kernel(p_ref, xf_ref, wpe_ref, bpe_ref, gblk_ref, bblk_ref,
                  w1_ref, b1_ref, w2_ref, b2_ref, seg_ref, wsx_ref, bs_ref,
                  gbase_ref, bbase_ref, nw_ref, nb_ref, wh_ref, bh_ref,
                  out_ref, pooled_acc, scout_acc):
    i = pl.program_id(0)
    bf16 = jnp.bfloat16
    pb = p_ref[:]

    xf = xf_ref[0]
    scout_cols = [jnp.sum(xf * wsx_ref[e, :][None, :], axis=1, keepdims=True)
                  for e in range(_E)]
    scout4 = jnp.concatenate(scout_cols, axis=1)

    h = jnp.dot(pb, wpe_ref[:], preferred_element_type=jnp.float32)
    h = h + bpe_ref[0, :]
    mu = jnp.mean(h, axis=-1, keepdims=True)
    var = jnp.mean((h - mu) * (h - mu), axis=-1, keepdims=True)
    hn = (h - mu) * jax.lax.rsqrt(var + _EPS) * gblk_ref[0, :] + bblk_ref[0, :]
    t = jnp.dot(hn.astype(bf16), w1_ref[:], preferred_element_type=jnp.float32)
    t = jax.nn.gelu(t + b1_ref[0, :])
    h = h + jnp.dot(t.astype(bf16), w2_ref[:], preferred_element_type=jnp.float32)
    h = h + b2_ref[0, :]

    pooled8 = jnp.dot(seg_ref[:], h, preferred_element_type=jnp.float32)
    pooled_acc[pl.ds(8 * i, 8), :] = pooled8
    scout_acc[pl.ds(8 * i, 8), :] = jnp.concatenate(
        [scout4, jnp.zeros_like(scout4)], axis=0)

    @pl.when(i == _G - 1)
    def _():
        gate = jax.nn.softmax(scout_acc[:] + bs_ref[0, :], axis=-1)
        pa = pooled_acc[:]
        mu2 = jnp.mean(pa, axis=-1, keepdims=True)
        var2 = jnp.mean((pa - mu2) * (pa - mu2), axis=-1, keepdims=True)
        xn = (pa - mu2) * jax.lax.rsqrt(var2 + _EPS)
        geff = gbase_ref[0, :] + jnp.dot(gate, nw_ref[:],
                                         preferred_element_type=jnp.float32)
        beff = bbase_ref[0, :] + jnp.dot(gate, nb_ref[:],
                                         preferred_element_type=jnp.float32)
        feat = xn * geff + beff
        logits = jnp.dot(feat, wh_ref[:], preferred_element_type=jnp.float32)
        out_ref[:] = logits + bh_ref[0, :]


def kernel(x, W_s, b_s, W_pe, b_pe, g_blk, b_blk, W1, b1, W2, b2,
           g_base, b_base, norms_w, norms_b, W_h, b_h):
    f32 = jnp.float32

    p = x.astype(jnp.bfloat16).reshape(_B, _C, _H // _P, _P, _W // _P, _P)
    p = p.transpose(0, 2, 4, 1, 3, 5).reshape(_B * _N, _P * _P * _C)

    wsx = W_s.T.reshape(_E, _C, 8, 1, 8, 1) * (1.0 / 784.0)
    wsx = jnp.broadcast_to(wsx, (_E, _C, 8, 28, 8, 28))
    wsx = wsx.reshape(_E, _C * _H * _W).astype(f32)
    xf3 = x.reshape(_G, _SPB, _C * _H * _W)

    row = jax.lax.broadcasted_iota(jnp.int32, (8, _BLK), 0)
    col = jax.lax.broadcasted_iota(jnp.int32, (8, _BLK), 1)
    seg = jnp.where(col // _N == row, 1.0 / _N, 0.0).astype(f32)

    full2 = lambda shp: pl.BlockSpec(shp, lambda i: (0, 0))
    logits64 = pl.pallas_call(
        _fused_kernel,
        grid=(_G,),
        in_specs=[
            pl.BlockSpec((_BLK, _D), lambda i: (i, 0)),
            pl.BlockSpec((1, _SPB, _C * _H * _W),
                         lambda i: (i, 0, 0)),
            full2((_D, _D)),
            full2((1, _D)),
            full2((1, _D)),
            full2((1, _D)),
            full2((_D, _DFF)),
            full2((1, _DFF)),
            full2((_DFF, _D)),
            full2((1, _D)),
            full2((8, _BLK)),
            full2((_E, _C * _H * _W)),
            full2((1, _E)),
            full2((1, _D)),
            full2((1, _D)),
            full2((_E, _D)),
            full2((_E, _D)),
            full2((_D, _NCLS)),
            full2((1, _NCLS)),
        ],
        out_specs=full2((_G * 8, _NCLS)),
        out_shape=jax.ShapeDtypeStruct((_G * 8, _NCLS), f32),
        scratch_shapes=[pltpu.VMEM((_G * 8, _D), f32),
                        pltpu.VMEM((_G * 8, _E), f32)],
    )(p, xf3, W_pe.astype(jnp.bfloat16), b_pe.reshape(1, _D),
      g_blk.reshape(1, _D), b_blk.reshape(1, _D),
      W1.astype(jnp.bfloat16), b1.reshape(1, _DFF), W2.astype(jnp.bfloat16),
      b2.reshape(1, _D), seg, wsx,
      b_s.reshape(1, _E), g_base.reshape(1, _D), b_base.reshape(1, _D),
      norms_w, norms_b, W_h, b_h.reshape(1, _NCLS))

    logits = logits64.reshape(_G, 8, _NCLS)[:, :_SPB].reshape(_B, _NCLS)
    return logits


# speedup vs baseline: 2.9043x; 1.1727x over previous
import jax
import jax.numpy as jnp
from jax.experimental import pallas as pl
from jax.experimental.pallas import tpu as pltpu

_B, _C, _H, _W = 32, 3, 224, 224
_P = 16
_N = (_H // _P) * (_W // _P)
_D = 768
_DFF = 3072
_E = 3
_NCLS = 1000
_SPB = 4
_BLK = _SPB * _N
_G = _B // _SPB
_EPS = 1e-6


def _fused_kernel(p_ref, xf_ref, wpe_ref, bpe_ref, gblk_ref, bblk_ref,
                  w1_ref, b1_ref, w2_ref, b2_ref, seg_ref, wsx_ref, bs_ref,
                  gbase_ref, bbase_ref, nw_ref, nb_ref, wh_ref, bh_ref,
                  out_ref, pooled_acc, scout_acc):
    i = pl.program_id(0)
    bf16 = jnp.bfloat16
    xb = p_ref[0]
    pb = xb.reshape(4, 3, 14, 16, 14, 16).transpose(0, 2, 4, 1, 3, 5)
    pb = pb.reshape(784, 768)

    xf = xf_ref[0]
    scout_cols = [jnp.sum(xf * wsx_ref[e, :][None, :], axis=1, keepdims=True)
                  for e in range(_E)]
    scout4 = jnp.concatenate(scout_cols, axis=1)

    h = jnp.dot(pb, wpe_ref[:], preferred_element_type=jnp.float32)
    h = h + bpe_ref[0, :]
    mu = jnp.mean(h, axis=-1, keepdims=True)
    var = jnp.mean((h - mu) * (h - mu), axis=-1, keepdims=True)
    hn = (h - mu) * jax.lax.rsqrt(var + _EPS) * gblk_ref[0, :] + bblk_ref[0, :]
    t = jnp.dot(hn.astype(bf16), w1_ref[:], preferred_element_type=jnp.float32)
    t = jax.nn.gelu(t + b1_ref[0, :])
    h = h + jnp.dot(t.astype(bf16), w2_ref[:], preferred_element_type=jnp.float32)
    h = h + b2_ref[0, :]

    pooled8 = jnp.dot(seg_ref[:], h, preferred_element_type=jnp.float32)
    pooled_acc[pl.ds(8 * i, 8), :] = pooled8
    scout_acc[pl.ds(8 * i, 8), :] = jnp.concatenate(
        [scout4, jnp.zeros_like(scout4)], axis=0)

    @pl.when(i == _G - 1)
    def _():
        gate = jax.nn.softmax(scout_acc[:] + bs_ref[0, :], axis=-1)
        pa = pooled_acc[:]
        mu2 = jnp.mean(pa, axis=-1, keepdims=True)
        var2 = jnp.mean((pa - mu2) * (pa - mu2), axis=-1, keepdims=True)
        xn = (pa - mu2) * jax.lax.rsqrt(var2 + _EPS)
        geff = gbase_ref[0, :] + jnp.dot(gate, nw_ref[:],
                                         preferred_element_type=jnp.float32)
        beff = bbase_ref[0, :] + jnp.dot(gate, nb_ref[:],
                                         preferred_element_type=jnp.float32)
        feat = xn * geff + beff
        logits = jnp.dot(feat, wh_ref[:], preferred_element_type=jnp.float32)
        out_ref[:] = logits + bh_ref[0, :]


def kernel(x, W_s, b_s, W_pe, b_pe, g_blk, b_blk, W1, b1, W2, b2,
           g_base, b_base, norms_w, norms_b, W_h, b_h):
    f32 = jnp.float32

    p = x.astype(jnp.bfloat16).reshape(_G, _SPB, _C, _H, _W)

    wsx = W_s.T.reshape(_E, _C, 8, 1, 8, 1) * (1.0 / 784.0)
    wsx = jnp.broadcast_to(wsx, (_E, _C, 8, 28, 8, 28))
    wsx = wsx.reshape(_E, _C * _H * _W).astype(f32)
    xf3 = x.reshape(_G, _SPB, _C * _H * _W)

    row = jax.lax.broadcasted_iota(jnp.int32, (8, _BLK), 0)
    col = jax.lax.broadcasted_iota(jnp.int32, (8, _BLK), 1)
    seg = jnp.where(col // _N == row, 1.0 / _N, 0.0).astype(f32)

    full2 = lambda shp: pl.BlockSpec(shp, lambda i: (0, 0))
    logits64 = pl.pallas_call(
        _fused_kernel,
        grid=(_G,),
        in_specs=[
            pl.BlockSpec((1, _SPB, _C, _H, _W),
                         lambda i: (i, 0, 0, 0, 0)),
            pl.BlockSpec((1, _SPB, _C * _H * _W),
                         lambda i: (i, 0, 0)),
            full2((_D, _D)),
            full2((1, _D)),
            full2((1, _D)),
            full2((1, _D)),
            full2((_D, _DFF)),
            full2((1, _DFF)),
            full2((_DFF, _D)),
            full2((1, _D)),
            full2((8, _BLK)),
            full2((_E, _C * _H * _W)),
            full2((1, _E)),
            full2((1, _D)),
            full2((1, _D)),
            full2((_E, _D)),
            full2((_E, _D)),
            full2((_D, _NCLS)),
            full2((1, _NCLS)),
        ],
        out_specs=full2((_G * 8, _NCLS)),
        out_shape=jax.ShapeDtypeStruct((_G * 8, _NCLS), f32),
        scratch_shapes=[pltpu.VMEM((_G * 8, _D), f32),
                        pltpu.VMEM((_G * 8, _E), f32)],
    )(p, xf3, W_pe.astype(jnp.bfloat16), b_pe.reshape(1, _D),
      g_blk.reshape(1, _D), b_blk.reshape(1, _D),
      W1.astype(jnp.bfloat16), b1.reshape(1, _DFF), W2.astype(jnp.bfloat16),
      b2.reshape(1, _D), seg, wsx,
      b_s.reshape(1, _E), g_base.reshape(1, _D), b_base.reshape(1, _D),
      norms_w, norms_b, W_h, b_h.reshape(1, _NCLS))

    logits = logits64.reshape(_G, 8, _NCLS)[:, :_SPB].reshape(_B, _NCLS)
    return logits


# speedup vs baseline: 4.7587x; 1.6385x over previous
import jax
import jax.numpy as jnp
from jax.experimental import pallas as pl
from jax.experimental.pallas import tpu as pltpu

_B, _C, _H, _W = 32, 3, 224, 224
_P = 16
_N = (_H // _P) * (_W // _P)
_D = 768
_DFF = 3072
_E = 3
_NCLS = 1000
_SPB = 4
_BLK = _SPB * _N
_G = _B // _SPB
_EPS = 1e-6


def _fused_kernel(p_ref, wpe_ref, bpe_ref, gblk_ref, bblk_ref,
                  w1_ref, b1_ref, w2_ref, b2_ref, seg_ref, wsx_ref, bs_ref,
                  gbase_ref, bbase_ref, nw_ref, nb_ref, wh_ref, bh_ref,
                  out_ref, pooled_acc, scout_acc):
    i = pl.program_id(0)
    bf16 = jnp.bfloat16
    xb = p_ref[0]
    pb = xb.astype(bf16).reshape(4, 3, 14, 16, 14, 16)
    pb = pb.transpose(0, 2, 4, 1, 3, 5).reshape(784, 768)

    scout_cols = []
    for e in range(_E):
        prod = xb * wsx_ref[e]
        scout_cols.append(jnp.sum(jnp.sum(prod, axis=2), axis=1)[:, None])
    scout4 = jnp.concatenate(scout_cols, axis=1)

    h = jnp.dot(pb, wpe_ref[:], preferred_element_type=jnp.float32)
    h = h + bpe_ref[0, :]
    mu = jnp.mean(h, axis=-1, keepdims=True)
    var = jnp.mean((h - mu) * (h - mu), axis=-1, keepdims=True)
    hn = (h - mu) * jax.lax.rsqrt(var + _EPS) * gblk_ref[0, :] + bblk_ref[0, :]
    t = jnp.dot(hn.astype(bf16), w1_ref[:], preferred_element_type=jnp.float32)
    t = jax.nn.gelu(t + b1_ref[0, :])
    h = h + jnp.dot(t.astype(bf16), w2_ref[:], preferred_element_type=jnp.float32)
    h = h + b2_ref[0, :]

    pooled8 = jnp.dot(seg_ref[:], h, preferred_element_type=jnp.float32)
    pooled_acc[pl.ds(8 * i, 8), :] = pooled8
    scout_acc[pl.ds(8 * i, 8), :] = jnp.concatenate(
        [scout4, jnp.zeros_like(scout4)], axis=0)

    @pl.when(i == _G - 1)
    def _():
        gate = jax.nn.softmax(scout_acc[:] + bs_ref[0, :], axis=-1)
        pa = pooled_acc[:]
        mu2 = jnp.mean(pa, axis=-1, keepdims=True)
        var2 = jnp.mean((pa - mu2) * (pa - mu2), axis=-1, keepdims=True)
        xn = (pa - mu2) * jax.lax.rsqrt(var2 + _EPS)
        geff = gbase_ref[0, :] + jnp.dot(gate, nw_ref[:],
                                         preferred_element_type=jnp.float32)
        beff = bbase_ref[0, :] + jnp.dot(gate, nb_ref[:],
                                         preferred_element_type=jnp.float32)
        feat = xn * geff + beff
        logits = jnp.dot(feat, wh_ref[:], preferred_element_type=jnp.float32)
        out_ref[:] = logits + bh_ref[0, :]


def kernel(x, W_s, b_s, W_pe, b_pe, g_blk, b_blk, W1, b1, W2, b2,
           g_base, b_base, norms_w, norms_b, W_h, b_h):
    f32 = jnp.float32

    p = x.reshape(_G, _SPB, _C * _H, _W)

    wsx = W_s.T.reshape(_E, _C, 8, 1, 8, 1) * (1.0 / 784.0)
    wsx = jnp.broadcast_to(wsx, (_E, _C, 8, 28, 8, 28))
    wsx = wsx.reshape(_E, _C * _H, _W).astype(f32)

    row = jax.lax.broadcasted_iota(jnp.int32, (8, _BLK), 0)
    col = jax.lax.broadcasted_iota(jnp.int32, (8, _BLK), 1)
    seg = jnp.where(col // _N == row, 1.0 / _N, 0.0).astype(f32)

    full2 = lambda shp: pl.BlockSpec(shp, lambda i: (0, 0))
    logits64 = pl.pallas_call(
        _fused_kernel,
        grid=(_G,),
        in_specs=[
            pl.BlockSpec((1, _SPB, _C * _H, _W),
                         lambda i: (i, 0, 0, 0)),
            full2((_D, _D)),
            full2((1, _D)),
            full2((1, _D)),
            full2((1, _D)),
            full2((_D, _DFF)),
            full2((1, _DFF)),
            full2((_DFF, _D)),
            full2((1, _D)),
            full2((8, _BLK)),
            pl.BlockSpec((_E, _C * _H, _W),
                         lambda i: (0, 0, 0)),
            full2((1, _E)),
            full2((1, _D)),
            full2((1, _D)),
            full2((_E, _D)),
            full2((_E, _D)),
            full2((_D, _NCLS)),
            full2((1, _NCLS)),
        ],
        out_specs=full2((_G * 8, _NCLS)),
        out_shape=jax.ShapeDtypeStruct((_G * 8, _NCLS), f32),
        scratch_shapes=[pltpu.VMEM((_G * 8, _D), f32),
                        pltpu.VMEM((_G * 8, _E), f32)],
    )(p, W_pe.astype(jnp.bfloat16), b_pe.reshape(1, _D),
      g_blk.reshape(1, _D), b_blk.reshape(1, _D),
      W1.astype(jnp.bfloat16), b1.reshape(1, _DFF), W2.astype(jnp.bfloat16),
      b2.reshape(1, _D), seg, wsx,
      b_s.reshape(1, _E), g_base.reshape(1, _D), b_base.reshape(1, _D),
      norms_w, norms_b, W_h, b_h.reshape(1, _NCLS))

    logits = logits64.reshape(_G, 8, _NCLS)[:, :_SPB].reshape(_B, _NCLS)
    return logits


# speedup vs baseline: 5.0390x; 1.0589x over previous
import jax
import jax.numpy as jnp
from jax.experimental import pallas as pl
from jax.experimental.pallas import tpu as pltpu

_B, _C, _H, _W = 32, 3, 224, 224
_P = 16
_N = (_H // _P) * (_W // _P)
_D = 768
_DFF = 3072
_E = 3
_NCLS = 1000
_SPB = 4
_BLK = _SPB * _N
_G = _B // _SPB
_EPS = 1e-6


def _fused_kernel(p_ref, wpe_ref, bpe_ref, gblk_ref, bblk_ref,
                  w1_ref, b1_ref, w2_ref, b2_ref, seg_ref, wsx_ref, bs_ref,
                  gbase_ref, bbase_ref, nw_ref, nb_ref, wh_ref, bh_ref,
                  out_ref, pooled_acc, scout_acc):
    i = pl.program_id(0)
    bf16 = jnp.bfloat16
    xb = p_ref[0]
    xc = xb.astype(bf16).reshape(12, 224, 224)
    x2 = jnp.swapaxes(xc, 1, 2)
    x3 = x2.reshape(12, 14, 16, 224)
    x4 = jnp.swapaxes(x3, 2, 3)
    x5 = x4.reshape(4, 3, 14, 14, 16, 16)
    pb = x5.transpose(0, 3, 2, 1, 4, 5).reshape(784, 768)

    scout_cols = []
    for e in range(_E):
        prod = xb * wsx_ref[e]
        scout_cols.append(jnp.sum(jnp.sum(prod, axis=2), axis=1)[:, None])
    scout4 = jnp.concatenate(scout_cols, axis=1)

    h = jnp.dot(pb, wpe_ref[:], preferred_element_type=jnp.float32)
    h = h + bpe_ref[0, :]
    mu = jnp.mean(h, axis=-1, keepdims=True)
    var = jnp.mean((h - mu) * (h - mu), axis=-1, keepdims=True)
    hn = (h - mu) * jax.lax.rsqrt(var + _EPS) * gblk_ref[0, :] + bblk_ref[0, :]
    t = jnp.dot(hn.astype(bf16), w1_ref[:], preferred_element_type=jnp.float32)
    t = jax.nn.gelu(t + b1_ref[0, :])
    h = h + jnp.dot(t.astype(bf16), w2_ref[:], preferred_element_type=jnp.float32)
    h = h + b2_ref[0, :]

    pooled8 = jnp.dot(seg_ref[:], h, preferred_element_type=jnp.float32)
    pooled_acc[pl.ds(8 * i, 8), :] = pooled8
    scout_acc[pl.ds(8 * i, 8), :] = jnp.concatenate(
        [scout4, jnp.zeros_like(scout4)], axis=0)

    @pl.when(i == _G - 1)
    def _():
        gate = jax.nn.softmax(scout_acc[:] + bs_ref[0, :], axis=-1)
        pa = pooled_acc[:]
        mu2 = jnp.mean(pa, axis=-1, keepdims=True)
        var2 = jnp.mean((pa - mu2) * (pa - mu2), axis=-1, keepdims=True)
        xn = (pa - mu2) * jax.lax.rsqrt(var2 + _EPS)
        geff = gbase_ref[0, :] + jnp.dot(gate, nw_ref[:],
                                         preferred_element_type=jnp.float32)
        beff = bbase_ref[0, :] + jnp.dot(gate, nb_ref[:],
                                         preferred_element_type=jnp.float32)
        feat = xn * geff + beff
        logits = jnp.dot(feat, wh_ref[:], preferred_element_type=jnp.float32)
        out_ref[:] = logits + bh_ref[0, :]


def kernel(x, W_s, b_s, W_pe, b_pe, g_blk, b_blk, W1, b1, W2, b2,
           g_base, b_base, norms_w, norms_b, W_h, b_h):
    f32 = jnp.float32

    p = x.reshape(_G, _SPB, _C * _H, _W)

    wsx = W_s.T.reshape(_E, _C, 8, 1, 8, 1) * (1.0 / 784.0)
    wsx = jnp.broadcast_to(wsx, (_E, _C, 8, 28, 8, 28))
    wsx = wsx.reshape(_E, _C * _H, _W).astype(f32)

    row = jax.lax.broadcasted_iota(jnp.int32, (8, _BLK), 0)
    col = jax.lax.broadcasted_iota(jnp.int32, (8, _BLK), 1)
    seg = jnp.where(col // _N == row, 1.0 / _N, 0.0).astype(f32)

    full2 = lambda shp: pl.BlockSpec(shp, lambda i: (0, 0))
    logits64 = pl.pallas_call(
        _fused_kernel,
        grid=(_G,),
        in_specs=[
            pl.BlockSpec((1, _SPB, _C * _H, _W),
                         lambda i: (i, 0, 0, 0)),
            full2((_D, _D)),
            full2((1, _D)),
            full2((1, _D)),
            full2((1, _D)),
            full2((_D, _DFF)),
            full2((1, _DFF)),
            full2((_DFF, _D)),
            full2((1, _D)),
            full2((8, _BLK)),
            pl.BlockSpec((_E, _C * _H, _W),
                         lambda i: (0, 0, 0)),
            full2((1, _E)),
            full2((1, _D)),
            full2((1, _D)),
            full2((_E, _D)),
            full2((_E, _D)),
            full2((_D, _NCLS)),
            full2((1, _NCLS)),
        ],
        out_specs=full2((_G * 8, _NCLS)),
        out_shape=jax.ShapeDtypeStruct((_G * 8, _NCLS), f32),
        scratch_shapes=[pltpu.VMEM((_G * 8, _D), f32),
                        pltpu.VMEM((_G * 8, _E), f32)],
    )(p, W_pe.astype(jnp.bfloat16), b_pe.reshape(1, _D),
      g_blk.reshape(1, _D), b_blk.reshape(1, _D),
      W1.astype(jnp.bfloat16), b1.reshape(1, _DFF), W2.astype(jnp.bfloat16),
      b2.reshape(1, _D), seg, wsx,
      b_s.reshape(1, _E), g_base.reshape(1, _D), b_base.reshape(1, _D),
      norms_w, norms_b, W_h, b_h.reshape(1, _NCLS))

    logits = logits64.reshape(_G, 8, _NCLS)[:, :_SPB].reshape(_B, _NCLS)
    return logits
